# baseline (speedup 1.0000x reference)
import functools

import jax
import jax.numpy as jnp
from jax import lax
from jax.experimental import pallas as pl
from jax.experimental.pallas import tpu as pltpu
from jax.experimental.pallas import tpu_sc as plsc

_info = plsc.get_sparse_core_info()
_NC, _NS, _L = _info.num_cores, _info.num_subcores, _info.num_lanes
_NW = _NC * _NS


def _masked_gather(flat_idx, table, *, chunk):
    total = flat_idx.shape[0]
    per_w = total // _NW
    nchunk = per_w // chunk
    mesh = plsc.VectorSubcoreMesh(core_axis_name="c", subcore_axis_name="s")

    @functools.partial(
        pl.kernel,
        mesh=mesh,
        compiler_params=pltpu.CompilerParams(needs_layout_passes=False),
        out_type=jax.ShapeDtypeStruct((total,), jnp.int32),
        scratch_types=[
            pltpu.VMEM((table.shape[0],), jnp.int32),
            pltpu.VMEM((chunk,), jnp.int32),
            pltpu.VMEM((chunk,), jnp.int32),
            pltpu.VMEM((chunk,), jnp.int32),
            pltpu.VMEM((chunk,), jnp.int32),
            pltpu.VMEM((chunk,), jnp.int32),
            pltpu.SemaphoreType.DMA,
            pltpu.SemaphoreType.DMA,
            pltpu.SemaphoreType.DMA,
            pltpu.SemaphoreType.DMA,
            pltpu.SemaphoreType.DMA,
        ],
    )
    def k(idx_hbm, table_hbm, out_hbm, table_v, i0, i1, i2, o0, o1,
          si0, si1, si2, so0, so1):
        wid = lax.axis_index("s") * _NC + lax.axis_index("c")
        base = wid * per_w
        ibufs, obufs = (i0, i1, i2), (o0, o1)
        isems, osems = (si0, si1, si2), (so0, so1)
        pltpu.sync_copy(table_hbm, table_v)

        def in_copy(c):
            b = c % 3
            return pltpu.make_async_copy(
                idx_hbm.at[pl.ds(base + c * chunk, chunk)], ibufs[b], isems[b])

        def out_copy(c):
            b = c & 1
            return pltpu.make_async_copy(
                obufs[b], out_hbm.at[pl.ds(base + c * chunk, chunk)], osems[b])

        for c in range(min(3, nchunk)):
            in_copy(c).start()
        for c in range(nchunk):
            in_copy(c).wait()
            if c >= 2:
                out_copy(c - 2).wait()
            ib, ob = ibufs[c % 3], obufs[c & 1]

            @plsc.parallel_loop(0, chunk, step=_L, unroll=16)
            def _gather_body(i):
                ob[pl.ds(i, _L)] = plsc.load_gather(table_v, [ib[pl.ds(i, _L)]])

            out_copy(c).start()
            if c + 3 < nchunk:
                in_copy(c + 3).start()
        for c in range(max(nchunk - 2, 0), nchunk):
            out_copy(c).wait()

    return k(flat_idx, table)


def kernel(atype, type_mask):
    nf, natom = atype.shape
    ntr, ntc = natom // 8, nf // 128
    flat = (atype.T.reshape(ntr, 8, ntc, 128)
            .transpose(0, 2, 1, 3).reshape(-1))
    flat_out = _masked_gather(flat, type_mask, chunk=20480)
    out = (flat_out.reshape(ntr, ntc, 8, 128)
           .transpose(0, 2, 1, 3).reshape(natom, nf).T)
    return out


# speedup vs baseline: 1.0439x; 1.0439x over previous
import functools

import jax
import jax.numpy as jnp
from jax import lax
from jax.experimental import pallas as pl
from jax.experimental.pallas import tpu as pltpu
from jax.experimental.pallas import tpu_sc as plsc

_info = plsc.get_sparse_core_info()
_NC, _NS, _L = _info.num_cores, _info.num_subcores, _info.num_lanes
_NW = _NC * _NS


def _masked_gather(flat_idx, table, *, chunk):
    total = flat_idx.shape[0]
    per_w = total // _NW
    nchunk = per_w // chunk
    mesh = plsc.VectorSubcoreMesh(core_axis_name="c", subcore_axis_name="s")

    @functools.partial(
        pl.kernel,
        mesh=mesh,
        compiler_params=pltpu.CompilerParams(needs_layout_passes=False),
        out_type=jax.ShapeDtypeStruct((total,), jnp.int32),
        scratch_types=[
            pltpu.VMEM((table.shape[0],), jnp.int32),
            pltpu.VMEM((chunk,), jnp.int32),
            pltpu.VMEM((chunk,), jnp.int32),
            pltpu.VMEM((chunk,), jnp.int32),
            pltpu.VMEM((chunk,), jnp.int32),
            pltpu.SemaphoreType.DMA,
            pltpu.SemaphoreType.DMA,
            pltpu.SemaphoreType.DMA,
            pltpu.SemaphoreType.DMA,
        ],
    )
    def k(idx_hbm, table_hbm, out_hbm, table_v, i0, i1, o0, o1,
          si0, si1, so0, so1):
        wid = lax.axis_index("s") * _NC + lax.axis_index("c")
        base = wid * per_w
        ibufs, obufs = (i0, i1), (o0, o1)
        isems, osems = (si0, si1), (so0, so1)
        pltpu.sync_copy(table_hbm, table_v)

        def in_copy(c, b):
            return pltpu.make_async_copy(
                idx_hbm.at[pl.ds(base + c * chunk, chunk)], ibufs[b], isems[b])

        def out_copy(c, b):
            return pltpu.make_async_copy(
                obufs[b], out_hbm.at[pl.ds(base + c * chunk, chunk)], osems[b])

        in_copy(0, 0).start()
        for c in range(nchunk):
            b = c & 1
            in_copy(c, b).wait()
            if c + 1 < nchunk:
                in_copy(c + 1, (c + 1) & 1).start()
            if c >= 2:
                out_copy(c - 2, b).wait()
            ib, ob = ibufs[b], obufs[b]

            @plsc.parallel_loop(0, chunk, step=_L, unroll=16)
            def _gather_body(i):
                ob[pl.ds(i, _L)] = plsc.load_gather(table_v, [ib[pl.ds(i, _L)]])

            out_copy(c, b).start()
        for c in range(max(nchunk - 2, 0), nchunk):
            out_copy(c, c & 1).wait()

    return k(flat_idx, table)


def kernel(atype, type_mask):
    nf, natom = atype.shape
    ntr, ntc = natom // 8, nf // 128
    flat = (atype.T.reshape(ntr, 8, ntc, 128)
            .transpose(0, 2, 1, 3).reshape(-1))
    flat_out = _masked_gather(flat, type_mask, chunk=25600)
    out = (flat_out.reshape(ntr, ntc, 8, 128)
           .transpose(0, 2, 1, 3).reshape(natom, nf).T)
    return out
